# baseline (speedup 1.0000x reference)
import functools

import jax
import jax.numpy as jnp
from jax import lax
from jax.experimental import pallas as pl
from jax.experimental.pallas import tpu as pltpu
from jax.experimental.pallas import tpu_sc as plsc

B, T, D, E, DFF = 1, 2048, 1024, 8, 4096
BLK = 1024
NB = 12
P = NB * BLK
FCH = 512
NF = DFF // FCH

NC, NS = 2, 16
NW = NC * NS
L = 16


def _router_body(x_ref, m_ref, gw_ref, mb_ref,
                 w12_ref, pos_ref, be_ref, act_ref, aux_ref):
    x = x_ref[...]
    gw = gw_ref[...]
    logits = lax.dot_general(x, gw, (((1,), (1,)), ((), ())),
                             preferred_element_type=jnp.float32)
    logits = logits + m_ref[...] * mb_ref[...]
    mx = jnp.max(logits, axis=1, keepdims=True)
    ex = jnp.exp(logits - mx)
    p = ex / jnp.sum(ex, axis=1, keepdims=True)

    eio = lax.broadcasted_iota(jnp.int32, p.shape, 1)
    p1 = jnp.max(p, axis=1, keepdims=True)
    i1 = jnp.min(jnp.where(p == p1, eio, E), axis=1, keepdims=True)
    pm = jnp.where(eio == i1, -jnp.inf, p)
    p2 = jnp.max(pm, axis=1, keepdims=True)
    i2 = jnp.min(jnp.where(pm == p2, eio, E), axis=1, keepdims=True)
    w12_ref[pl.ds(0, T), :] = p1
    w12_ref[pl.ds(T, T), :] = p2

    imp = jnp.sum(p, axis=0, keepdims=True)
    target = jnp.float32(T) / jnp.float32(E)
    aux_ref[...] = jnp.mean((imp - target) ** 2, keepdims=True).reshape(1, 1)

    m1 = (eio == i1).astype(jnp.int32)
    m2 = (eio == i2).astype(jnp.int32)

    def cumsum0(a):
        sh = 1
        while sh < T:
            a = a + jnp.concatenate(
                [jnp.zeros((sh, E), a.dtype), a[:-sh]], axis=0)
            sh *= 2
        return a

    cs1 = cumsum0(m1)
    cs2 = cumsum0(m2)
    count1 = cs1[T - 1:T, :]
    counts = count1 + cs2[T - 1:T, :]
    rank1 = jnp.sum(cs1 * m1, axis=1, keepdims=True) - 1
    rank2 = jnp.sum((cs2 + count1) * m2, axis=1, keepdims=True) - 1

    nb = (counts + (BLK - 1)) // BLK
    incl = nb
    for sh in (1, 2, 4):
        incl = incl + jnp.concatenate(
            [jnp.zeros((1, sh), incl.dtype), incl[:, :-sh]], axis=1)
    po_b = incl - nb
    po = po_b * BLK
    pos1 = jnp.sum(m1 * po, axis=1, keepdims=True) + rank1
    pos2 = jnp.sum(m2 * po, axis=1, keepdims=True) + rank2
    pos_ref[pl.ds(0, T), :] = pos1
    pos_ref[pl.ds(T, T), :] = pos2

    bio = lax.broadcasted_iota(jnp.int32, (NB, E), 0)
    eio_b = lax.broadcasted_iota(jnp.int32, (NB, E), 1)
    in_e = (bio >= po_b) & (bio < po_b + nb)
    be_raw = jnp.sum(jnp.where(in_e, eio_b, 0), axis=1, keepdims=True)
    act = jnp.sum(in_e.astype(jnp.int32), axis=1, keepdims=True)
    eio_1 = lax.broadcasted_iota(jnp.int32, (1, E), 1)
    elast = jnp.max(jnp.where(nb > 0, eio_1, 0))
    be_ref[...] = jnp.where(act > 0, be_raw, elast)
    act_ref[...] = act


def _router(x, m, gate_W, mass_bias):
    return pl.pallas_call(
        _router_body,
        out_shape=(
            jax.ShapeDtypeStruct((2 * T, 1), jnp.float32),
            jax.ShapeDtypeStruct((2 * T, 1), jnp.int32),
            jax.ShapeDtypeStruct((NB, 1), jnp.int32),
            jax.ShapeDtypeStruct((NB, 1), jnp.int32),
            jax.ShapeDtypeStruct((1, 1), jnp.float32),
        ),
    )(x, m, gate_W, mass_bias.reshape(1, E))


def _sc_mesh():
    return plsc.VectorSubcoreMesh(core_axis_name="c", subcore_axis_name="s")


@functools.lru_cache(maxsize=None)
def _build_meta_scatter():
    @functools.partial(
        pl.kernel,
        out_type=[
            jax.ShapeDtypeStruct((P,), jnp.int32),
            jax.ShapeDtypeStruct((P,), jnp.float32),
        ],
        mesh=_sc_mesh(),
        compiler_params=pltpu.CompilerParams(needs_layout_passes=False),
        scratch_types=[
            pltpu.VMEM((2 * T,), jnp.int32),
            pltpu.VMEM((2 * T,), jnp.float32),
            pltpu.VMEM((P,), jnp.int32),
            pltpu.VMEM((P,), jnp.float32),
        ],
    )
    def _meta_scatter(pos_hbm, w_hbm, tok_out, ws_out,
                      pos_v, w_v, tok_v, wsort_v):
        wid = lax.axis_index("s") * NC + lax.axis_index("c")

        @pl.when(wid == 0)
        def _():
            pltpu.sync_copy(pos_hbm, pos_v)
            pltpu.sync_copy(w_hbm, w_v)
            zi = jnp.zeros((L,), jnp.int32)
            zf = jnp.zeros((L,), jnp.float32)
            for i in range(P // L):
                tok_v[pl.ds(i * L, L)] = zi
                wsort_v[pl.ds(i * L, L)] = zf
            for i in range(2 * T // L):
                idx = pos_v[pl.ds(i * L, L)]
                val = w_v[pl.ds(i * L, L)]
                plsc.store_scatter(wsort_v, [idx], val)
                tok = lax.iota(jnp.int32, L) + ((i * L) % T)
                plsc.store_scatter(tok_v, [idx], tok)
            pltpu.sync_copy(tok_v, tok_out)
            pltpu.sync_copy(wsort_v, ws_out)

    return _meta_scatter


def _ffn_body(be_s, act_s, src_ref, ws_ref, x_ref, W1_ref, b1_ref, W2_ref,
              b2_ref, out_ref, xg_ref):
    b = pl.program_id(0)
    f = pl.program_id(1)

    @pl.when(act_s[b] != 0)
    def _():
        @pl.when(f == 0)
        def _dispatch():
            tio = lax.broadcasted_iota(jnp.int32, (BLK, T), 1)
            g = (tio == src_ref[...]).astype(jnp.bfloat16)
            xg_ref[...] = lax.dot_general(
                g, x_ref[...], (((1,), (0,)), ((), ())),
                preferred_element_type=jnp.float32).astype(jnp.bfloat16)

        w1c = W1_ref[0].astype(jnp.bfloat16)
        h = lax.dot_general(xg_ref[...], w1c, (((1,), (1,)), ((), ())),
                            preferred_element_type=jnp.float32)
        h = h + b1_ref[0]
        h = 0.5 * h * (1.0 + lax.erf(h * 0.7071067811865476))
        hw = (h * ws_ref[...]).astype(jnp.bfloat16)
        w2c = W2_ref[0].astype(jnp.bfloat16)
        acc = lax.dot_general(hw, w2c, (((1,), (1,)), ((), ())),
                              preferred_element_type=jnp.float32)

        @pl.when(f == 0)
        def _init():
            out_ref[...] = acc + ws_ref[...] * b2_ref[0]

        @pl.when(f != 0)
        def _accum():
            out_ref[...] += acc


def _grouped_ffn(be, act, src, ws, x_bf, W1, b1, W2, b2):
    grid_spec = pltpu.PrefetchScalarGridSpec(
        num_scalar_prefetch=2,
        grid=(NB, NF),
        in_specs=[
            pl.BlockSpec((BLK, 1), lambda b, f, be, act: (b, 0)),
            pl.BlockSpec((BLK, 1), lambda b, f, be, act: (b, 0)),
            pl.BlockSpec((T, D), lambda b, f, be, act: (0, 0)),
            pl.BlockSpec((1, FCH, D), lambda b, f, be, act: (be[b], f, 0)),
            pl.BlockSpec((1, 1, FCH),
                         lambda b, f, be, act: (be[b] * NF + f, 0, 0)),
            pl.BlockSpec((1, D, FCH), lambda b, f, be, act: (be[b], 0, f)),
            pl.BlockSpec((1, 1, D), lambda b, f, be, act: (be[b], 0, 0)),
        ],
        out_specs=pl.BlockSpec((BLK, D), lambda b, f, be, act: (b, 0)),
        scratch_shapes=[pltpu.VMEM((BLK, D), jnp.bfloat16)],
    )
    return pl.pallas_call(
        _ffn_body,
        grid_spec=grid_spec,
        out_shape=jax.ShapeDtypeStruct((P, D), jnp.float32),
    )(be, act, src.reshape(P, 1), ws.reshape(P, 1), x_bf, W1,
      b1.reshape(E * NF, 1, FCH), W2, b2.reshape(E, 1, D))


_CCH = 32


@functools.lru_cache(maxsize=None)
def _build_combine():
    @functools.partial(
        pl.kernel,
        out_type=jax.ShapeDtypeStruct((T, D), jnp.float32),
        mesh=_sc_mesh(),
        scratch_types=[
            pltpu.VMEM((_CCH,), jnp.int32),
            pltpu.VMEM((_CCH,), jnp.int32),
            pltpu.VMEM((_CCH, D), jnp.float32),
            pltpu.VMEM((_CCH, D), jnp.float32),
            pltpu.SemaphoreType.DMA,
            pltpu.SemaphoreType.DMA,
        ],
    )
    def _combine(ys_hbm, pos_hbm, out_hbm,
                 idx1_v, idx2_v, r1_v, r2_v, sem1, sem2):
        wid = lax.axis_index("s") * NC + lax.axis_index("c")
        per_w = T // NW
        for c in range(per_w // _CCH):
            base = wid * per_w + c * _CCH
            pltpu.sync_copy(pos_hbm.at[pl.ds(base, _CCH)], idx1_v)
            pltpu.sync_copy(pos_hbm.at[pl.ds(T + base, _CCH)], idx2_v)
            cp1 = pltpu.async_copy(ys_hbm.at[idx1_v], r1_v, sem1)
            cp2 = pltpu.async_copy(ys_hbm.at[idx2_v], r2_v, sem2)
            cp1.wait()
            cp2.wait()

            def add_row(t, _):
                for ch in range(D // L):
                    sl = pl.ds(ch * L, L)
                    r1_v[t, sl] = r1_v[t, sl] + r2_v[t, sl]
                return 0

            lax.fori_loop(0, _CCH, add_row, 0)
            pltpu.sync_copy(r1_v, out_hbm.at[pl.ds(base, _CCH)])

    return _combine


def kernel(hidden_states, mass, gate_W, mass_bias, W1, b1, W2, b2):
    x = hidden_states.reshape(T, D)
    m = mass.reshape(T, 1)

    w_flat, pos_flat, be, act, aux = _router(x, m, gate_W, mass_bias)
    pos_flat = pos_flat.reshape(2 * T)
    w_flat = w_flat.reshape(2 * T)

    src_tok, w_sorted = _build_meta_scatter()(pos_flat, w_flat)

    ys = _grouped_ffn(be.reshape(NB), act.reshape(NB), src_tok, w_sorted,
                      x.astype(jnp.bfloat16), W1, b1, W2, b2)

    out = _build_combine()(ys, pos_flat)
    return out.reshape(hidden_states.shape), aux[0, 0]


# speedup vs baseline: 1.1011x; 1.1011x over previous
import functools

import jax
import jax.numpy as jnp
from jax import lax
from jax.experimental import pallas as pl
from jax.experimental.pallas import tpu as pltpu
from jax.experimental.pallas import tpu_sc as plsc

B, T, D, E, DFF = 1, 2048, 1024, 8, 4096
BLK = 512
NB = 16
P = NB * BLK
FCH = 512
NF = DFF // FCH

NC, NS = 2, 16
NW = NC * NS
L = 16


def _router_body(x_ref, m_ref, gw_ref, mb_ref,
                 w12_ref, pos_ref, be_ref, act_ref, aux_ref):
    x = x_ref[...]
    gw = gw_ref[...]
    logits = lax.dot_general(x, gw, (((1,), (1,)), ((), ())),
                             preferred_element_type=jnp.float32)
    logits = logits + m_ref[...] * mb_ref[...]
    mx = jnp.max(logits, axis=1, keepdims=True)
    ex = jnp.exp(logits - mx)
    p = ex / jnp.sum(ex, axis=1, keepdims=True)

    eio = lax.broadcasted_iota(jnp.int32, p.shape, 1)
    p1 = jnp.max(p, axis=1, keepdims=True)
    i1 = jnp.min(jnp.where(p == p1, eio, E), axis=1, keepdims=True)
    pm = jnp.where(eio == i1, -jnp.inf, p)
    p2 = jnp.max(pm, axis=1, keepdims=True)
    i2 = jnp.min(jnp.where(pm == p2, eio, E), axis=1, keepdims=True)
    w12_ref[pl.ds(0, T), :] = p1
    w12_ref[pl.ds(T, T), :] = p2

    imp = jnp.sum(p, axis=0, keepdims=True)
    target = jnp.float32(T) / jnp.float32(E)
    aux_ref[...] = jnp.mean((imp - target) ** 2, keepdims=True).reshape(1, 1)

    m1 = (eio == i1).astype(jnp.int32)
    m2 = (eio == i2).astype(jnp.int32)

    def cumsum0(a):
        sh = 1
        while sh < T:
            a = a + jnp.concatenate(
                [jnp.zeros((sh, E), a.dtype), a[:-sh]], axis=0)
            sh *= 2
        return a

    cs1 = cumsum0(m1)
    cs2 = cumsum0(m2)
    count1 = cs1[T - 1:T, :]
    counts = count1 + cs2[T - 1:T, :]
    rank1 = jnp.sum(cs1 * m1, axis=1, keepdims=True) - 1
    rank2 = jnp.sum((cs2 + count1) * m2, axis=1, keepdims=True) - 1

    nb = (counts + (BLK - 1)) // BLK
    incl = nb
    for sh in (1, 2, 4):
        incl = incl + jnp.concatenate(
            [jnp.zeros((1, sh), incl.dtype), incl[:, :-sh]], axis=1)
    po_b = incl - nb
    po = po_b * BLK
    pos1 = jnp.sum(m1 * po, axis=1, keepdims=True) + rank1
    pos2 = jnp.sum(m2 * po, axis=1, keepdims=True) + rank2
    pos_ref[pl.ds(0, T), :] = pos1
    pos_ref[pl.ds(T, T), :] = pos2

    bio = lax.broadcasted_iota(jnp.int32, (NB, E), 0)
    eio_b = lax.broadcasted_iota(jnp.int32, (NB, E), 1)
    in_e = (bio >= po_b) & (bio < po_b + nb)
    be_raw = jnp.sum(jnp.where(in_e, eio_b, 0), axis=1, keepdims=True)
    act = jnp.sum(in_e.astype(jnp.int32), axis=1, keepdims=True)
    eio_1 = lax.broadcasted_iota(jnp.int32, (1, E), 1)
    elast = jnp.max(jnp.where(nb > 0, eio_1, 0))
    be_ref[...] = jnp.where(act > 0, be_raw, elast)
    act_ref[...] = act


def _router(x, m, gate_W, mass_bias):
    return pl.pallas_call(
        _router_body,
        out_shape=(
            jax.ShapeDtypeStruct((2 * T, 1), jnp.float32),
            jax.ShapeDtypeStruct((2 * T, 1), jnp.int32),
            jax.ShapeDtypeStruct((NB, 1), jnp.int32),
            jax.ShapeDtypeStruct((NB, 1), jnp.int32),
            jax.ShapeDtypeStruct((1, 1), jnp.float32),
        ),
    )(x, m, gate_W, mass_bias.reshape(1, E))


def _sc_mesh():
    return plsc.VectorSubcoreMesh(core_axis_name="c", subcore_axis_name="s")


@functools.lru_cache(maxsize=None)
def _build_meta_scatter():
    @functools.partial(
        pl.kernel,
        out_type=[
            jax.ShapeDtypeStruct((P,), jnp.int32),
            jax.ShapeDtypeStruct((P,), jnp.float32),
        ],
        mesh=_sc_mesh(),
        compiler_params=pltpu.CompilerParams(needs_layout_passes=False),
        scratch_types=[
            pltpu.VMEM((2 * T,), jnp.int32),
            pltpu.VMEM((2 * T,), jnp.float32),
            pltpu.VMEM((P,), jnp.int32),
            pltpu.VMEM((P,), jnp.float32),
        ],
    )
    def _meta_scatter(pos_hbm, w_hbm, tok_out, ws_out,
                      pos_v, w_v, tok_v, wsort_v):
        wid = lax.axis_index("s") * NC + lax.axis_index("c")

        @pl.when(wid == 0)
        def _():
            pltpu.sync_copy(pos_hbm, pos_v)
            pltpu.sync_copy(w_hbm, w_v)
            zi = jnp.zeros((L,), jnp.int32)
            zf = jnp.zeros((L,), jnp.float32)
            for i in range(P // L):
                tok_v[pl.ds(i * L, L)] = zi
                wsort_v[pl.ds(i * L, L)] = zf
            for i in range(2 * T // L):
                idx = pos_v[pl.ds(i * L, L)]
                val = w_v[pl.ds(i * L, L)]
                plsc.store_scatter(wsort_v, [idx], val)
                tok = lax.iota(jnp.int32, L) + ((i * L) % T)
                plsc.store_scatter(tok_v, [idx], tok)
            pltpu.sync_copy(tok_v, tok_out)
            pltpu.sync_copy(wsort_v, ws_out)

    return _meta_scatter


def _ffn_body(be_s, act_s, src_ref, ws_ref, x_ref, W1_ref, b1_ref, W2_ref,
              b2_ref, out_ref, xg_ref):
    b = pl.program_id(0)
    f = pl.program_id(1)

    @pl.when(act_s[b] != 0)
    def _():
        @pl.when(f == 0)
        def _dispatch():
            tio = lax.broadcasted_iota(jnp.int32, (BLK, T), 1)
            g = (tio == src_ref[...]).astype(jnp.bfloat16)
            xg_ref[...] = lax.dot_general(
                g, x_ref[...], (((1,), (0,)), ((), ())),
                preferred_element_type=jnp.float32).astype(jnp.bfloat16)

        w1c = W1_ref[0].astype(jnp.bfloat16)
        h = lax.dot_general(xg_ref[...], w1c, (((1,), (1,)), ((), ())),
                            preferred_element_type=jnp.float32)
        h = h + b1_ref[0]
        h = 0.5 * h * (1.0 + lax.erf(h * 0.7071067811865476))
        hw = (h * ws_ref[...]).astype(jnp.bfloat16)
        w2c = W2_ref[0].astype(jnp.bfloat16)
        acc = lax.dot_general(hw, w2c, (((1,), (1,)), ((), ())),
                              preferred_element_type=jnp.float32)

        @pl.when(f == 0)
        def _init():
            out_ref[...] = acc + ws_ref[...] * b2_ref[0]

        @pl.when(f != 0)
        def _accum():
            out_ref[...] += acc


def _grouped_ffn(be, act, src, ws, x_bf, W1, b1, W2, b2):
    grid_spec = pltpu.PrefetchScalarGridSpec(
        num_scalar_prefetch=2,
        grid=(NB, NF),
        in_specs=[
            pl.BlockSpec((BLK, 1), lambda b, f, be, act: (b, 0)),
            pl.BlockSpec((BLK, 1), lambda b, f, be, act: (b, 0)),
            pl.BlockSpec((T, D), lambda b, f, be, act: (0, 0)),
            pl.BlockSpec((1, FCH, D), lambda b, f, be, act: (be[b], f, 0)),
            pl.BlockSpec((1, 1, FCH),
                         lambda b, f, be, act: (be[b] * NF + f, 0, 0)),
            pl.BlockSpec((1, D, FCH), lambda b, f, be, act: (be[b], 0, f)),
            pl.BlockSpec((1, 1, D), lambda b, f, be, act: (be[b], 0, 0)),
        ],
        out_specs=pl.BlockSpec((BLK, D), lambda b, f, be, act: (b, 0)),
        scratch_shapes=[pltpu.VMEM((BLK, D), jnp.bfloat16)],
    )
    return pl.pallas_call(
        _ffn_body,
        grid_spec=grid_spec,
        out_shape=jax.ShapeDtypeStruct((P, D), jnp.float32),
    )(be, act, src.reshape(P, 1), ws.reshape(P, 1), x_bf, W1,
      b1.reshape(E * NF, 1, FCH), W2, b2.reshape(E, 1, D))


_CCH = 32


@functools.lru_cache(maxsize=None)
def _build_combine():
    @functools.partial(
        pl.kernel,
        out_type=jax.ShapeDtypeStruct((T, D), jnp.float32),
        mesh=_sc_mesh(),
        scratch_types=[
            pltpu.VMEM((_CCH,), jnp.int32),
            pltpu.VMEM((_CCH,), jnp.int32),
            pltpu.VMEM((_CCH, D), jnp.float32),
            pltpu.VMEM((_CCH, D), jnp.float32),
            pltpu.SemaphoreType.DMA,
            pltpu.SemaphoreType.DMA,
        ],
    )
    def _combine(ys_hbm, pos_hbm, out_hbm,
                 idx1_v, idx2_v, r1_v, r2_v, sem1, sem2):
        wid = lax.axis_index("s") * NC + lax.axis_index("c")
        per_w = T // NW
        for c in range(per_w // _CCH):
            base = wid * per_w + c * _CCH
            pltpu.sync_copy(pos_hbm.at[pl.ds(base, _CCH)], idx1_v)
            pltpu.sync_copy(pos_hbm.at[pl.ds(T + base, _CCH)], idx2_v)
            cp1 = pltpu.async_copy(ys_hbm.at[idx1_v], r1_v, sem1)
            cp2 = pltpu.async_copy(ys_hbm.at[idx2_v], r2_v, sem2)
            cp1.wait()
            cp2.wait()

            def add_row(t, _):
                for ch in range(D // L):
                    sl = pl.ds(ch * L, L)
                    r1_v[t, sl] = r1_v[t, sl] + r2_v[t, sl]
                return 0

            lax.fori_loop(0, _CCH, add_row, 0)
            pltpu.sync_copy(r1_v, out_hbm.at[pl.ds(base, _CCH)])

    return _combine


def kernel(hidden_states, mass, gate_W, mass_bias, W1, b1, W2, b2):
    x = hidden_states.reshape(T, D)
    m = mass.reshape(T, 1)

    w_flat, pos_flat, be, act, aux = _router(x, m, gate_W, mass_bias)
    pos_flat = pos_flat.reshape(2 * T)
    w_flat = w_flat.reshape(2 * T)

    src_tok, w_sorted = _build_meta_scatter()(pos_flat, w_flat)

    ys = _grouped_ffn(be.reshape(NB), act.reshape(NB), src_tok, w_sorted,
                      x.astype(jnp.bfloat16), W1, b1, W2, b2)

    out = _build_combine()(ys, pos_flat)
    return out.reshape(hidden_states.shape), aux[0, 0]
